# Initial kernel scaffold; baseline (speedup 1.0000x reference)
import jax
import jax.numpy as jnp
from jax.experimental import pallas as pl


def kernel(x, edge_index, edge_attr, batch, W1, b1, W2, b2, W3, b3, lin1_W, lin1_b, lin2_W, lin2_b):
    raise NotImplementedError("write your pallas kernel here")


# speedup vs baseline: 4.2721x; 4.2721x over previous
import dataclasses
import functools

import jax
import jax.numpy as jnp
from jax import lax
from jax.experimental import pallas as pl
from jax.experimental.pallas import tpu as pltpu
from jax.experimental.pallas import tpu_sc as plsc

N = 10000
E = 320000
F = 128
G = 128
CLS = 10

NC = 2
NS = 16
NW = NC * NS
EPW = E // NW
EB = 80
NBAT = EPW // EB
NP = 10240
RPT = NP // NS
ZR = 128
GPT = G // NS

@functools.lru_cache(maxsize=1)
def _sc_params():
    cp = pltpu.CompilerParams()
    if "needs_layout_passes" in pltpu.CompilerParams.__dataclass_fields__:
        cp = dataclasses.replace(cp, needs_layout_passes=False)
    return cp


@functools.lru_cache(maxsize=1)
def _sc_mesh():
    return plsc.VectorSubcoreMesh(
        core_axis_name="c", subcore_axis_name="s", num_cores=NC, num_subcores=NS
    )

_f32 = jnp.float32
_i32 = jnp.int32


def _wid():
    return lax.axis_index("c") * NS + lax.axis_index("s")


def _fill_zeros(ref, rows, width):
    zv = jnp.zeros((16,), _f32)

    @pl.loop(0, rows)
    def _(r):
        for k in range(width // 16):
            ref[r, pl.ds(k * 16, 16)] = zv


def _deg_body(src_hbm, w_hbm, out_hbm, acc, val, idx_s, wv, zbuf):
    core = lax.axis_index("c")
    sub = lax.axis_index("s")
    _fill_zeros(zbuf, ZR, F)

    @pl.loop(0, RPT, step=ZR)
    def _(r0):
        pltpu.sync_copy(zbuf, acc.at[pl.ds(sub * RPT + r0, ZR)])

    plsc.subcore_barrier()
    base = _wid() * EPW

    @pl.loop(0, NBAT)
    def _(ib):
        off = base + ib * EB
        pltpu.sync_copy(src_hbm.at[pl.ds(off, EB)], idx_s)
        pltpu.sync_copy(w_hbm.at[pl.ds(off, EB)], wv)

        @pl.loop(0, EB)
        def _(e):
            c = plsc.load_gather(wv, [jnp.full((16,), e, _i32)])
            for k in range(F // 16):
                val[e, pl.ds(k * 16, 16)] = c

        pltpu.sync_copy(val, acc.at[idx_s], add=True)

    plsc.subcore_barrier()
    rows = pl.ds(sub * RPT, RPT)
    pltpu.sync_copy(acc.at[rows], out_hbm.at[core, rows])


def _deg_call(src, w):
    return pl.kernel(
        _deg_body,
        out_type=jax.ShapeDtypeStruct((NC, NP, F), _f32),
        mesh=_sc_mesh(),
        compiler_params=_sc_params(),
        scratch_types=[
            pltpu.VMEM_SHARED((NP, F), _f32),
            pltpu.VMEM((EB, F), _f32),
            pltpu.VMEM((EB,), _i32),
            pltpu.VMEM((EB,), _f32),
            pltpu.VMEM((ZR, F), _f32),
        ],
    )(src, w)


def _matvec_body(hs_hbm, src_hbm, dst_hbm, w_hbm, out_hbm,
                 acc, rows, idx_s, idx_d, wv, zbuf, sem):
    core = lax.axis_index("c")
    sub = lax.axis_index("s")
    _fill_zeros(zbuf, ZR, F)

    @pl.loop(0, RPT, step=ZR)
    def _(r0):
        pltpu.sync_copy(zbuf, acc.at[pl.ds(sub * RPT + r0, ZR)])

    plsc.subcore_barrier()
    base = _wid() * EPW

    @pl.loop(0, NBAT)
    def _(ib):
        off = base + ib * EB
        pltpu.sync_copy(src_hbm.at[pl.ds(off, EB)], idx_s)
        pltpu.sync_copy(dst_hbm.at[pl.ds(off, EB)], idx_d)
        pltpu.sync_copy(w_hbm.at[pl.ds(off, EB)], wv)
        pltpu.async_copy(hs_hbm.at[idx_s], rows, sem).wait()

        @pl.loop(0, EB)
        def _(e):
            c = plsc.load_gather(wv, [jnp.full((16,), e, _i32)])
            for k in range(F // 16):
                sl = pl.ds(k * 16, 16)
                rows[e, sl] = rows[e, sl] * c

        pltpu.sync_copy(rows, acc.at[idx_d], add=True)

    plsc.subcore_barrier()
    rsl = pl.ds(sub * RPT, RPT)
    pltpu.sync_copy(acc.at[rsl], out_hbm.at[core, rsl])


def _matvec_call(hs, src, dst, w):
    return pl.kernel(
        _matvec_body,
        out_type=jax.ShapeDtypeStruct((NC, NP, F), _f32),
        mesh=_sc_mesh(),
        compiler_params=_sc_params(),
        scratch_types=[
            pltpu.VMEM_SHARED((NP, F), _f32),
            pltpu.VMEM((EB, F), _f32),
            pltpu.VMEM((EB,), _i32),
            pltpu.VMEM((EB,), _i32),
            pltpu.VMEM((EB,), _f32),
            pltpu.VMEM((ZR, F), _f32),
            pltpu.SemaphoreType.DMA,
        ],
    )(hs, src, dst, w)


NCHUNK = N // EB


def _pool_body(h_hbm, batch_hbm, outs_hbm, outc_hbm,
               accs, accc, hbuf, ones_v, bidx, zs, zc):
    core = lax.axis_index("c")
    sub = lax.axis_index("s")
    wid = _wid()
    _fill_zeros(zs, GPT, F)
    _fill_zeros(zc, GPT, F)
    ov = jnp.ones((16,), _f32)

    @pl.loop(0, EB)
    def _(r):
        for k in range(F // 16):
            ones_v[r, pl.ds(k * 16, 16)] = ov

    gsl = pl.ds(sub * GPT, GPT)
    pltpu.sync_copy(zs, accs.at[gsl])
    pltpu.sync_copy(zc, accc.at[gsl])
    plsc.subcore_barrier()

    @pl.loop(0, NCHUNK)
    def _(ic):
        @pl.when(lax.rem(ic, NW) == wid)
        def _():
            off = ic * EB
            pltpu.sync_copy(h_hbm.at[pl.ds(off, EB)], hbuf)
            pltpu.sync_copy(batch_hbm.at[pl.ds(off, EB)], bidx)
            pltpu.sync_copy(hbuf, accs.at[bidx], add=True)
            pltpu.sync_copy(ones_v, accc.at[bidx], add=True)

    plsc.subcore_barrier()
    pltpu.sync_copy(accs.at[gsl], outs_hbm.at[core, gsl])
    pltpu.sync_copy(accc.at[gsl], outc_hbm.at[core, gsl])


def _pool_call(h, batch):
    return pl.kernel(
        _pool_body,
        out_type=(
            jax.ShapeDtypeStruct((NC, G, F), _f32),
            jax.ShapeDtypeStruct((NC, G, F), _f32),
        ),
        mesh=_sc_mesh(),
        compiler_params=_sc_params(),
        scratch_types=[
            pltpu.VMEM_SHARED((G, F), _f32),
            pltpu.VMEM_SHARED((G, F), _f32),
            pltpu.VMEM((EB, F), _f32),
            pltpu.VMEM((EB, F), _f32),
            pltpu.VMEM((EB,), _i32),
            pltpu.VMEM((GPT, F), _f32),
            pltpu.VMEM((GPT, F), _f32),
        ],
    )(h, batch)


BLK = 1000
NBLK = N // BLK


def _dis_tc_body(degp_ref, x_ref, dis_ref, s0_ref):
    deg = degp_ref[0][:, :16] + degp_ref[1][:, :16]
    ok = deg > 0.0
    dis = jnp.where(ok, lax.rsqrt(jnp.where(ok, deg, 1.0)), 0.0)
    dis_ref[...] = dis
    s0_ref[...] = x_ref[...] * dis[:, :1]


def _dis_call(degp, x):
    return pl.pallas_call(
        _dis_tc_body,
        grid=(NBLK,),
        in_specs=[
            pl.BlockSpec((NC, BLK, F), lambda i: (0, i, 0)),
            pl.BlockSpec((BLK, F), lambda i: (i, 0)),
        ],
        out_specs=[
            pl.BlockSpec((BLK, 16), lambda i: (i, 0)),
            pl.BlockSpec((BLK, F), lambda i: (i, 0)),
        ],
        out_shape=[
            jax.ShapeDtypeStruct((N, 16), _f32),
            jax.ShapeDtypeStruct((N, F), _f32),
        ],
    )(degp, x)


def _k1_tc_body(p_ref, h_ref, dis_ref, w0_ref, w1_ref, acc_ref, s1_ref):
    d = dis_ref[...][:, :1]
    t1 = -(p_ref[0] + p_ref[1]) * d
    acc_ref[...] = (
        jnp.dot(h_ref[...], w0_ref[...], preferred_element_type=_f32)
        + jnp.dot(t1, w1_ref[...], preferred_element_type=_f32)
    )
    s1_ref[...] = t1 * d


def _k1_call(p, h, dis, w0, w1):
    return pl.pallas_call(
        _k1_tc_body,
        grid=(NBLK,),
        in_specs=[
            pl.BlockSpec((NC, BLK, F), lambda i: (0, i, 0)),
            pl.BlockSpec((BLK, F), lambda i: (i, 0)),
            pl.BlockSpec((BLK, 16), lambda i: (i, 0)),
            pl.BlockSpec((F, F), lambda i: (0, 0)),
            pl.BlockSpec((F, F), lambda i: (0, 0)),
        ],
        out_specs=[
            pl.BlockSpec((BLK, F), lambda i: (i, 0)),
            pl.BlockSpec((BLK, F), lambda i: (i, 0)),
        ],
        out_shape=[
            jax.ShapeDtypeStruct((N, F), _f32),
            jax.ShapeDtypeStruct((N, F), _f32),
        ],
    )(p, h, dis, w0, w1)


def _k2_tc_body(p_ref, h_ref, acc_ref, dis_ref, w2_ref, b_ref,
                out_ref, sout_ref):
    d = dis_ref[...][:, :1]
    t2 = -2.0 * (p_ref[0] + p_ref[1]) * d - h_ref[...]
    o = acc_ref[...] + jnp.dot(t2, w2_ref[...], preferred_element_type=_f32)
    o = jnp.maximum(o + b_ref[...], 0.0)
    out_ref[...] = o
    sout_ref[...] = o * d


def _k2_call(p, h, acc, dis, w2, b):
    return pl.pallas_call(
        _k2_tc_body,
        grid=(NBLK,),
        in_specs=[
            pl.BlockSpec((NC, BLK, F), lambda i: (0, i, 0)),
            pl.BlockSpec((BLK, F), lambda i: (i, 0)),
            pl.BlockSpec((BLK, F), lambda i: (i, 0)),
            pl.BlockSpec((BLK, 16), lambda i: (i, 0)),
            pl.BlockSpec((F, F), lambda i: (0, 0)),
            pl.BlockSpec((1, F), lambda i: (0, 0)),
        ],
        out_specs=[
            pl.BlockSpec((BLK, F), lambda i: (i, 0)),
            pl.BlockSpec((BLK, F), lambda i: (i, 0)),
        ],
        out_shape=[
            jax.ShapeDtypeStruct((N, F), _f32),
            jax.ShapeDtypeStruct((N, F), _f32),
        ],
    )(p, h, acc, dis, w2, b)


def _head_tc_body(s_ref, c_ref, w1_ref, b1_ref, w2_ref, b2_ref, out_ref):
    cnt = (c_ref[0] + c_ref[1])[:, :1]
    g = (s_ref[0] + s_ref[1]) / jnp.maximum(cnt, 1.0)
    z1 = jnp.maximum(
        jnp.dot(g, w1_ref[...], preferred_element_type=_f32) + b1_ref[...], 0.0
    )
    z2 = jnp.dot(z1, w2_ref[...], preferred_element_type=_f32) + b2_ref[...]
    mask = lax.broadcasted_iota(_i32, (G, F), 1) < CLS
    neg = jnp.full((G, F), -1e30, _f32)
    z2m = jnp.where(mask, z2, neg)
    m = jnp.max(z2m, axis=1, keepdims=True)
    ex = jnp.where(mask, jnp.exp(z2m - m), 0.0)
    ls = jnp.log(jnp.sum(ex, axis=1, keepdims=True)) + m
    out_ref[...] = z2 - ls


def _head_call(s, c, w1, b1, w2p, b2p):
    return pl.pallas_call(
        _head_tc_body,
        grid=(1,),
        in_specs=[
            pl.BlockSpec((NC, G, F), lambda i: (0, 0, 0)),
            pl.BlockSpec((NC, G, F), lambda i: (0, 0, 0)),
            pl.BlockSpec((F, F), lambda i: (0, 0)),
            pl.BlockSpec((1, F), lambda i: (0, 0)),
            pl.BlockSpec((F, F), lambda i: (0, 0)),
            pl.BlockSpec((1, F), lambda i: (0, 0)),
        ],
        out_specs=pl.BlockSpec((G, F), lambda i: (0, 0)),
        out_shape=jax.ShapeDtypeStruct((G, F), _f32),
    )(s, c, w1, b1, w2p, b2p)


@jax.jit
def _run(x, edge_index, edge_attr, batch, W1, b1, W2, b2, W3, b3,
         lin1_W, lin1_b, lin2_W, lin2_b):
    src = edge_index[0]
    dst = edge_index[1]
    degp = _deg_call(src, edge_attr)
    dis, s = _dis_call(degp, x)
    h = x
    for (W, b) in ((W1, b1), (W2, b2), (W3, b3)):
        p1 = _matvec_call(s, src, dst, edge_attr)
        acc, s1 = _k1_call(p1, h, dis, W[0], W[1])
        p2 = _matvec_call(s1, src, dst, edge_attr)
        h, s = _k2_call(p2, h, acc, dis, W[2], b.reshape(1, F))
    S, C = _pool_call(h, batch)
    w2p = jnp.pad(lin2_W, ((0, 0), (0, F - CLS)))
    b2p = jnp.pad(lin2_b, (0, F - CLS)).reshape(1, F)
    outp = _head_call(S, C, lin1_W, lin1_b.reshape(1, F), w2p, b2p)
    return outp[:, :CLS]


def kernel(x, edge_index, edge_attr, batch, W1, b1, W2, b2, W3, b3,
           lin1_W, lin1_b, lin2_W, lin2_b):
    return _run(x, edge_index, edge_attr, batch, W1, b1, W2, b2, W3, b3,
                lin1_W, lin1_b, lin2_W, lin2_b)
